# baseline (speedup 1.0000x reference)
import functools

import jax
import jax.numpy as jnp
from jax import lax
from jax.experimental import pallas as pl
from jax.experimental.pallas import tpu as pltpu
from jax.experimental.pallas import tpu_sc as plsc

N = 8192
D = 256
K = 8192
BN = 2048
BK = 512
COMMITMENT_COST = 0.25
ENTROPY_WEIGHT = 0.1

_NC = 2
_NS = 16
_NW = _NC * _NS
_BPW = N // _NW


def _argmin_body(z_ref, w_ref, w2_ref, z2_ref, idx_ref):
    z = z_ref[...]
    z2 = z2_ref[...]

    def step(kb, carry):
        mv, mi = carry
        wb = w_ref[pl.ds(kb * BK, BK), :]
        m = lax.dot_general(z, wb, (((1,), (1,)), ((), ())),
                            preferred_element_type=jnp.float32)
        w2b = w2_ref[:, pl.ds(kb * BK, BK)]
        d2 = (z2 + w2b) - 2.0 * m
        dist = jnp.sqrt(jnp.maximum(d2, 0.0))
        bmin = jnp.min(dist, axis=1, keepdims=True)
        ii = lax.broadcasted_iota(jnp.int32, (BN, BK), 1) + kb * BK
        bidx = jnp.min(jnp.where(dist == bmin, ii, K), axis=1, keepdims=True)
        upd = bmin < mv
        return jnp.where(upd, bmin, mv), jnp.where(upd, bidx, mi)

    mv0 = jnp.full((BN, 1), jnp.inf, jnp.float32)
    mi0 = jnp.zeros((BN, 1), jnp.int32)
    _, mi = lax.fori_loop(0, K // BK, step, (mv0, mi0))
    idx_ref[...] = mi


_argmin_call = pl.pallas_call(
    _argmin_body,
    grid=(N // BN,),
    in_specs=[
        pl.BlockSpec((BN, D), lambda i: (i, 0)),
        pl.BlockSpec((K, D), lambda i: (0, 0)),
        pl.BlockSpec((1, K), lambda i: (0, 0)),
        pl.BlockSpec((BN, 1), lambda i: (i, 0)),
    ],
    out_specs=pl.BlockSpec((BN, 1), lambda i: (i, 0)),
    out_shape=jax.ShapeDtypeStruct((N, 1), jnp.int32),
)


@functools.cache
def _make_gather_zq():
    mesh = plsc.VectorSubcoreMesh(core_axis_name="c", subcore_axis_name="s")

    @functools.partial(
        pl.kernel,
        mesh=mesh,
        out_type=jax.ShapeDtypeStruct((N, D), jnp.float32),
        scratch_types=[
            pltpu.VMEM((_BPW,), jnp.int32),
            pltpu.VMEM((_BPW, D), jnp.float32),
            pltpu.SemaphoreType.DMA,
        ],
    )
    def _gather_zq(w_hbm, idx_hbm, out_hbm, idx_v, rows_v, sem):
        wid = lax.axis_index("s") * _NC + lax.axis_index("c")
        base = wid * _BPW
        pltpu.sync_copy(idx_hbm.at[pl.ds(base, _BPW)], idx_v)
        pltpu.async_copy(w_hbm.at[idx_v], rows_v, sem).wait()
        pltpu.sync_copy(rows_v, out_hbm.at[pl.ds(base, _BPW)])

    return _gather_zq


_HB = 256


def _loss_body(ze_ref, zq_ref, idx_ref, st_ref, scal_ref, counts_ref, acc_ref):
    i = pl.program_id(0)
    nsteps = pl.num_programs(0)

    @pl.when(i == 0)
    def _init():
        acc_ref[0] = 0.0
        counts_ref[...] = jnp.zeros((1, K), jnp.float32)

    ze = ze_ref[...]
    zq = zq_ref[...]
    diff = zq - ze
    st_ref[...] = ze + diff
    acc_ref[0] += jnp.sum(diff * diff)

    def hstep(c, acc):
        col = idx_ref[pl.ds(c * _HB, _HB), :]
        oh = jnp.where(col == lax.broadcasted_iota(jnp.int32, (_HB, K), 1),
                       1.0, 0.0)
        return acc + jnp.sum(oh, axis=0, keepdims=True)

    counts_ref[...] = lax.fori_loop(0, BN // _HB, hstep, counts_ref[...])

    @pl.when(i == nsteps - 1)
    def _fin():
        p = counts_ref[...] * (1.0 / N) + 1e-10
        ent = -jnp.sum(p * jnp.log(p))
        cb = acc_ref[0] / (N * D)
        scal_ref[0] = cb
        scal_ref[1] = COMMITMENT_COST * cb
        scal_ref[2] = -ENTROPY_WEIGHT * (ent / jnp.log(float(K)))
        scal_ref[3] = jnp.exp(ent)


_loss_call = pl.pallas_call(
    _loss_body,
    grid=(N // BN,),
    in_specs=[
        pl.BlockSpec((BN, D), lambda i: (i, 0)),
        pl.BlockSpec((BN, D), lambda i: (i, 0)),
        pl.BlockSpec((BN, 1), lambda i: (i, 0)),
    ],
    out_specs=[
        pl.BlockSpec((BN, D), lambda i: (i, 0)),
        pl.BlockSpec(memory_space=pltpu.SMEM),
    ],
    out_shape=[
        jax.ShapeDtypeStruct((N, D), jnp.float32),
        jax.ShapeDtypeStruct((4,), jnp.float32),
    ],
    scratch_shapes=[
        pltpu.VMEM((1, K), jnp.float32),
        pltpu.SMEM((1,), jnp.float32),
    ],
)


def kernel(z_e, W):
    z2 = jnp.sum(z_e * z_e, axis=-1, keepdims=True)
    w2 = jnp.sum(W * W, axis=-1)[None, :]
    idx_col = _argmin_call(z_e, W, w2, z2)
    idx = idx_col.reshape(N)
    z_q = _make_gather_zq()(W, idx)
    z_q_st, scal = _loss_call(z_e, z_q, idx_col)
    return (z_q_st, idx, scal[0], scal[1], scal[2], scal[3])


# speedup vs baseline: 1.0938x; 1.0938x over previous
import functools

import jax
import jax.numpy as jnp
from jax import lax
from jax.experimental import pallas as pl
from jax.experimental.pallas import tpu as pltpu
from jax.experimental.pallas import tpu_sc as plsc

N = 8192
D = 256
K = 8192
BN = 2048
BK = 512
COMMITMENT_COST = 0.25
ENTROPY_WEIGHT = 0.1

_NC = 2
_NS = 16
_NW = _NC * _NS
_BPW = N // _NW


def _bits(x):
    return lax.bitcast_convert_type(x, jnp.int32)


def _flt(i):
    return lax.bitcast_convert_type(i, jnp.float32)


def _argmin_body(z_ref, w_ref, w2_ref, z2_ref, idx_ref):
    z = z_ref[...]
    z2 = z2_ref[...]

    def step(kb, carry):
        sv, mi = carry
        wb = w_ref[pl.ds(kb * BK, BK), :]
        m = lax.dot_general(wb, z, (((1,), (1,)), ((), ())),
                            preferred_element_type=jnp.float32)
        w2b = w2_ref[pl.ds(kb * BK, BK), :]
        d2 = (z2 + w2b) - 2.0 * m
        dist = jnp.sqrt(jnp.maximum(d2, 0.0))
        s = jnp.min(dist, axis=0, keepdims=True)
        ii = lax.broadcasted_iota(jnp.int32, (BK, BN), 0)
        bidx = jnp.min(jnp.where(dist == s, ii, BK), axis=0,
                       keepdims=True) + kb * BK
        upd = s < sv
        return jnp.where(upd, s, sv), jnp.where(upd, bidx, mi)

    sv0 = jnp.full((1, BN), jnp.inf, jnp.float32)
    mi0 = jnp.zeros((1, BN), jnp.int32)
    _, mi = lax.fori_loop(0, K // BK, step, (sv0, mi0))
    idx_ref[...] = mi


_argmin_call = pl.pallas_call(
    _argmin_body,
    grid=(N // BN,),
    in_specs=[
        pl.BlockSpec((BN, D), lambda i: (i, 0)),
        pl.BlockSpec((K, D), lambda i: (0, 0)),
        pl.BlockSpec((K, 1), lambda i: (0, 0)),
        pl.BlockSpec((1, BN), lambda i: (0, i)),
    ],
    out_specs=pl.BlockSpec((1, BN), lambda i: (0, i)),
    out_shape=jax.ShapeDtypeStruct((1, N), jnp.int32),
)


@functools.cache
def _make_gather_zq():
    mesh = plsc.VectorSubcoreMesh(core_axis_name="c", subcore_axis_name="s")

    @functools.partial(
        pl.kernel,
        mesh=mesh,
        out_type=jax.ShapeDtypeStruct((N, D), jnp.float32),
        scratch_types=[
            pltpu.VMEM((_BPW,), jnp.int32),
            pltpu.VMEM((_BPW, D), jnp.float32),
            pltpu.SemaphoreType.DMA,
        ],
    )
    def _gather_zq(w_hbm, idx_hbm, out_hbm, idx_v, rows_v, sem):
        wid = lax.axis_index("s") * _NC + lax.axis_index("c")
        base = wid * _BPW
        pltpu.sync_copy(idx_hbm.at[pl.ds(base, _BPW)], idx_v)
        pltpu.async_copy(w_hbm.at[idx_v], rows_v, sem).wait()
        pltpu.sync_copy(rows_v, out_hbm.at[pl.ds(base, _BPW)])

    return _gather_zq


_HB = 256


def _loss_body(ze_ref, zq_ref, idx_ref, st_ref, scal_ref, counts_ref, acc_ref):
    i = pl.program_id(0)
    nsteps = pl.num_programs(0)

    @pl.when(i == 0)
    def _init():
        acc_ref[0] = 0.0
        counts_ref[...] = jnp.zeros((1, K), jnp.float32)

    ze = ze_ref[...]
    zq = zq_ref[...]
    diff = zq - ze
    st_ref[...] = ze + diff
    acc_ref[0] += jnp.sum(diff * diff)

    def hstep(c, acc):
        col = idx_ref[pl.ds(c * _HB, _HB), :]
        oh = jnp.where(col == lax.broadcasted_iota(jnp.int32, (_HB, K), 1),
                       1.0, 0.0)
        return acc + jnp.sum(oh, axis=0, keepdims=True)

    counts_ref[...] = lax.fori_loop(0, BN // _HB, hstep, counts_ref[...])

    @pl.when(i == nsteps - 1)
    def _fin():
        p = counts_ref[...] * (1.0 / N) + 1e-10
        ent = -jnp.sum(p * jnp.log(p))
        cb = acc_ref[0] / (N * D)
        scal_ref[0] = cb
        scal_ref[1] = COMMITMENT_COST * cb
        scal_ref[2] = -ENTROPY_WEIGHT * (ent / jnp.log(float(K)))
        scal_ref[3] = jnp.exp(ent)


_loss_call = pl.pallas_call(
    _loss_body,
    grid=(N // BN,),
    in_specs=[
        pl.BlockSpec((BN, D), lambda i: (i, 0)),
        pl.BlockSpec((BN, D), lambda i: (i, 0)),
        pl.BlockSpec((BN, 1), lambda i: (i, 0)),
    ],
    out_specs=[
        pl.BlockSpec((BN, D), lambda i: (i, 0)),
        pl.BlockSpec(memory_space=pltpu.SMEM),
    ],
    out_shape=[
        jax.ShapeDtypeStruct((N, D), jnp.float32),
        jax.ShapeDtypeStruct((4,), jnp.float32),
    ],
    scratch_shapes=[
        pltpu.VMEM((1, K), jnp.float32),
        pltpu.SMEM((1,), jnp.float32),
    ],
)


def kernel(z_e, W):
    z2 = jnp.sum(z_e * z_e, axis=-1)[None, :]
    w2 = jnp.sum(W * W, axis=-1, keepdims=True)
    idx_row = _argmin_call(z_e, W, w2, z2)
    idx = idx_row.reshape(N)
    z_q = _make_gather_zq()(W, idx)
    z_q_st, scal = _loss_call(z_e, z_q, idx_row.reshape(N, 1))
    return (z_q_st, idx, scal[0], scal[1], scal[2], scal[3])


# speedup vs baseline: 1.2248x; 1.1197x over previous
import functools

import jax
import jax.numpy as jnp
from jax import lax
from jax.experimental import pallas as pl
from jax.experimental.pallas import tpu as pltpu
from jax.experimental.pallas import tpu_sc as plsc

N = 8192
D = 256
K = 8192
BN = 2048
BK = 512
COMMITMENT_COST = 0.25
ENTROPY_WEIGHT = 0.1

_NC = 2
_NS = 16
_NW = _NC * _NS
_BPW = N // _NW


def _bits(x):
    return lax.bitcast_convert_type(x, jnp.int32)


def _flt(i):
    return lax.bitcast_convert_type(i, jnp.float32)


def _argmin_body(z_ref, w_ref, w2_ref, z2_ref, idx_ref):
    z = z_ref[...]
    z2 = z2_ref[...]

    def step(kb, carry):
        sv, mi = carry
        wb = w_ref[pl.ds(kb * BK, BK), :]
        m = lax.dot_general(wb, z, (((1,), (1,)), ((), ())),
                            preferred_element_type=jnp.float32)
        w2b = w2_ref[pl.ds(kb * BK, BK), :]
        d2 = (z2 + w2b) - 2.0 * m
        dist = jnp.sqrt(jnp.maximum(d2, 0.0))
        s = jnp.min(dist, axis=0, keepdims=True)
        ii = lax.broadcasted_iota(jnp.int32, (BK, BN), 0)
        bidx = jnp.min(jnp.where(dist == s, ii, BK), axis=0,
                       keepdims=True) + kb * BK
        upd = s < sv
        return jnp.where(upd, s, sv), jnp.where(upd, bidx, mi)

    sv0 = jnp.full((1, BN), jnp.inf, jnp.float32)
    mi0 = jnp.zeros((1, BN), jnp.int32)
    _, mi = lax.fori_loop(0, K // BK, step, (sv0, mi0))
    idx_ref[...] = mi


_argmin_call = pl.pallas_call(
    _argmin_body,
    grid=(N // BN,),
    in_specs=[
        pl.BlockSpec((BN, D), lambda i: (i, 0)),
        pl.BlockSpec((K, D), lambda i: (0, 0)),
        pl.BlockSpec((K, 1), lambda i: (0, 0)),
        pl.BlockSpec((1, BN), lambda i: (0, i)),
    ],
    out_specs=pl.BlockSpec((1, BN), lambda i: (0, i)),
    out_shape=jax.ShapeDtypeStruct((1, N), jnp.int32),
)


_SCH = _BPW // 128


@functools.cache
def _make_gather_zq():
    mesh = plsc.VectorSubcoreMesh(core_axis_name="c", subcore_axis_name="s")

    @functools.partial(
        pl.kernel,
        mesh=mesh,
        out_type=(
            jax.ShapeDtypeStruct((N, D), jnp.float32),
            jax.ShapeDtypeStruct((_NC, K), jnp.float32),
        ),
        scratch_types=[
            pltpu.VMEM((_BPW,), jnp.int32),
            pltpu.VMEM((_BPW, D), jnp.float32),
            pltpu.VMEM((_SCH, 128), jnp.int32),
            pltpu.VMEM((128,), jnp.float32),
            pltpu.VMEM_SHARED((K,), jnp.float32),
            pltpu.SemaphoreType.DMA,
        ],
    )
    def _gather_zq(w_hbm, idx_hbm, zeros_hbm, out_hbm, cnt_hbm,
                   idx_v, rows_v, idxc_v, ones_v, counts_sh, sem):
        cid = lax.axis_index("c")
        sid = lax.axis_index("s")
        wid = sid * _NC + cid
        base = wid * _BPW
        pltpu.sync_copy(idx_hbm.at[pl.ds(base, _BPW)], idx_v)
        pltpu.async_copy(w_hbm.at[idx_v], rows_v, sem).wait()
        pltpu.sync_copy(rows_v, out_hbm.at[pl.ds(base, _BPW)])

        for j in range(_SCH):
            pltpu.sync_copy(idx_hbm.at[pl.ds(base + j * 128, 128)], idxc_v.at[j])
        for j in range(8):
            ones_v[pl.ds(j * 16, 16)] = jnp.full((16,), 1.0, jnp.float32)

        @pl.when(sid == 0)
        def _zero():
            pltpu.sync_copy(zeros_hbm.at[cid], counts_sh)

        plsc.subcore_barrier()
        for j in range(_SCH):
            pltpu.sync_copy(ones_v, counts_sh.at[idxc_v.at[j]], add=True)
        plsc.subcore_barrier()

        @pl.when(sid == 0)
        def _out():
            pltpu.sync_copy(counts_sh, cnt_hbm.at[cid])

    return _gather_zq


def _loss_body(ze_ref, zq_ref, cnt_ref, st_ref, scal_ref, acc_ref):
    i = pl.program_id(0)
    nsteps = pl.num_programs(0)

    @pl.when(i == 0)
    def _init():
        acc_ref[0] = 0.0

    ze = ze_ref[...]
    zq = zq_ref[...]
    diff = zq - ze
    st_ref[...] = ze + diff
    acc_ref[0] += jnp.sum(diff * diff)

    @pl.when(i == nsteps - 1)
    def _fin():
        cnt = cnt_ref[0, :] + cnt_ref[1, :]
        p = cnt * (1.0 / N) + 1e-10
        ent = -jnp.sum(p * jnp.log(p))
        cb = acc_ref[0] / (N * D)
        scal_ref[0] = cb
        scal_ref[1] = COMMITMENT_COST * cb
        scal_ref[2] = -ENTROPY_WEIGHT * (ent / jnp.log(float(K)))
        scal_ref[3] = jnp.exp(ent)


_loss_call = pl.pallas_call(
    _loss_body,
    grid=(N // BN,),
    in_specs=[
        pl.BlockSpec((BN, D), lambda i: (i, 0)),
        pl.BlockSpec((BN, D), lambda i: (i, 0)),
        pl.BlockSpec((_NC, K), lambda i: (0, 0)),
    ],
    out_specs=[
        pl.BlockSpec((BN, D), lambda i: (i, 0)),
        pl.BlockSpec(memory_space=pltpu.SMEM),
    ],
    out_shape=[
        jax.ShapeDtypeStruct((N, D), jnp.float32),
        jax.ShapeDtypeStruct((4,), jnp.float32),
    ],
    scratch_shapes=[
        pltpu.SMEM((1,), jnp.float32),
    ],
)


def kernel(z_e, W):
    z2 = jnp.sum(z_e * z_e, axis=-1)[None, :]
    w2 = jnp.sum(W * W, axis=-1, keepdims=True)
    idx_row = _argmin_call(z_e, W, w2, z2)
    idx = idx_row.reshape(N)
    zeros2k = jnp.zeros((_NC, K), jnp.float32)
    z_q, counts = _make_gather_zq()(W, idx, zeros2k)
    z_q_st, scal = _loss_call(z_e, z_q, counts)
    return (z_q_st, idx, scal[0], scal[1], scal[2], scal[3])


# speedup vs baseline: 1.2279x; 1.0025x over previous
import functools

import jax
import jax.numpy as jnp
from jax import lax
from jax.experimental import pallas as pl
from jax.experimental.pallas import tpu as pltpu
from jax.experimental.pallas import tpu_sc as plsc

N = 8192
D = 256
K = 8192
BN = 2048
BK = 512
COMMITMENT_COST = 0.25
ENTROPY_WEIGHT = 0.1

_NC = 2
_NS = 16
_NW = _NC * _NS
_BPW = N // _NW


def _bits(x):
    return lax.bitcast_convert_type(x, jnp.int32)


def _flt(i):
    return lax.bitcast_convert_type(i, jnp.float32)


def _argmin_body(z_ref, w_ref, w2_ref, z2_ref, idx_ref):
    z = z_ref[...]
    z2 = z2_ref[...]

    def step(kb, carry):
        sv, mi = carry
        wb = w_ref[pl.ds(kb * BK, BK), :]
        m = lax.dot_general(wb, z, (((1,), (1,)), ((), ())),
                            preferred_element_type=jnp.float32)
        w2b = w2_ref[pl.ds(kb * BK, BK), :]
        d2 = (z2 + w2b) - 2.0 * m
        dist = jnp.sqrt(jnp.maximum(d2, 0.0))
        t = dist
        while t.shape[0] > 8:
            h = t.shape[0] // 2
            t = jnp.minimum(t[:h], t[h:])
        s = jnp.min(t, axis=0, keepdims=True)
        ii = lax.broadcasted_iota(jnp.int32, (BK, BN), 0)
        v = jnp.where(dist == s, ii, BK)
        while v.shape[0] > 8:
            h = v.shape[0] // 2
            v = jnp.minimum(v[:h], v[h:])
        bidx = jnp.min(v, axis=0, keepdims=True) + kb * BK
        upd = s < sv
        return jnp.where(upd, s, sv), jnp.where(upd, bidx, mi)

    sv0 = jnp.full((1, BN), jnp.inf, jnp.float32)
    mi0 = jnp.zeros((1, BN), jnp.int32)
    _, mi = lax.fori_loop(0, K // BK, step, (sv0, mi0))
    idx_ref[...] = mi


_argmin_call = pl.pallas_call(
    _argmin_body,
    grid=(N // BN,),
    in_specs=[
        pl.BlockSpec((BN, D), lambda i: (i, 0)),
        pl.BlockSpec((K, D), lambda i: (0, 0)),
        pl.BlockSpec((K, 1), lambda i: (0, 0)),
        pl.BlockSpec((1, BN), lambda i: (0, i)),
    ],
    out_specs=pl.BlockSpec((1, BN), lambda i: (0, i)),
    out_shape=jax.ShapeDtypeStruct((1, N), jnp.int32),
)


_SCH = _BPW // 128


@functools.cache
def _make_gather_zq():
    mesh = plsc.VectorSubcoreMesh(core_axis_name="c", subcore_axis_name="s")

    @functools.partial(
        pl.kernel,
        mesh=mesh,
        out_type=(
            jax.ShapeDtypeStruct((N, D), jnp.float32),
            jax.ShapeDtypeStruct((_NC, K), jnp.float32),
        ),
        scratch_types=[
            pltpu.VMEM((_BPW,), jnp.int32),
            pltpu.VMEM((_BPW, D), jnp.float32),
            pltpu.VMEM((_SCH, 128), jnp.int32),
            pltpu.VMEM((128,), jnp.float32),
            pltpu.VMEM_SHARED((K,), jnp.float32),
            pltpu.SemaphoreType.DMA,
        ],
    )
    def _gather_zq(w_hbm, idx_hbm, zeros_hbm, out_hbm, cnt_hbm,
                   idx_v, rows_v, idxc_v, ones_v, counts_sh, sem):
        cid = lax.axis_index("c")
        sid = lax.axis_index("s")
        wid = sid * _NC + cid
        base = wid * _BPW
        pltpu.sync_copy(idx_hbm.at[pl.ds(base, _BPW)], idx_v)
        pltpu.async_copy(w_hbm.at[idx_v], rows_v, sem).wait()
        pltpu.sync_copy(rows_v, out_hbm.at[pl.ds(base, _BPW)])

        for j in range(_SCH):
            pltpu.sync_copy(idx_hbm.at[pl.ds(base + j * 128, 128)], idxc_v.at[j])
        for j in range(8):
            ones_v[pl.ds(j * 16, 16)] = jnp.full((16,), 1.0, jnp.float32)

        @pl.when(sid == 0)
        def _zero():
            pltpu.sync_copy(zeros_hbm.at[cid], counts_sh)

        plsc.subcore_barrier()
        for j in range(_SCH):
            pltpu.sync_copy(ones_v, counts_sh.at[idxc_v.at[j]], add=True)
        plsc.subcore_barrier()

        @pl.when(sid == 0)
        def _out():
            pltpu.sync_copy(counts_sh, cnt_hbm.at[cid])

    return _gather_zq


def _loss_body(ze_ref, zq_ref, cnt_ref, st_ref, scal_ref, acc_ref):
    i = pl.program_id(0)
    nsteps = pl.num_programs(0)

    @pl.when(i == 0)
    def _init():
        acc_ref[0] = 0.0

    ze = ze_ref[...]
    zq = zq_ref[...]
    diff = zq - ze
    st_ref[...] = ze + diff
    acc_ref[0] += jnp.sum(diff * diff)

    @pl.when(i == nsteps - 1)
    def _fin():
        cnt = cnt_ref[0, :] + cnt_ref[1, :]
        p = cnt * (1.0 / N) + 1e-10
        ent = -jnp.sum(p * jnp.log(p))
        cb = acc_ref[0] / (N * D)
        scal_ref[0] = cb
        scal_ref[1] = COMMITMENT_COST * cb
        scal_ref[2] = -ENTROPY_WEIGHT * (ent / jnp.log(float(K)))
        scal_ref[3] = jnp.exp(ent)


_loss_call = pl.pallas_call(
    _loss_body,
    grid=(N // BN,),
    in_specs=[
        pl.BlockSpec((BN, D), lambda i: (i, 0)),
        pl.BlockSpec((BN, D), lambda i: (i, 0)),
        pl.BlockSpec((_NC, K), lambda i: (0, 0)),
    ],
    out_specs=[
        pl.BlockSpec((BN, D), lambda i: (i, 0)),
        pl.BlockSpec(memory_space=pltpu.SMEM),
    ],
    out_shape=[
        jax.ShapeDtypeStruct((N, D), jnp.float32),
        jax.ShapeDtypeStruct((4,), jnp.float32),
    ],
    scratch_shapes=[
        pltpu.SMEM((1,), jnp.float32),
    ],
)


def kernel(z_e, W):
    z2 = jnp.sum(z_e * z_e, axis=-1)[None, :]
    w2 = jnp.sum(W * W, axis=-1, keepdims=True)
    idx_row = _argmin_call(z_e, W, w2, z2)
    idx = idx_row.reshape(N)
    zeros2k = jnp.zeros((_NC, K), jnp.float32)
    z_q, counts = _make_gather_zq()(W, idx, zeros2k)
    z_q_st, scal = _loss_call(z_e, z_q, counts)
    return (z_q_st, idx, scal[0], scal[1], scal[2], scal[3])


# speedup vs baseline: 1.2589x; 1.0253x over previous
import functools

import jax
import jax.numpy as jnp
import numpy as np
from jax import lax
from jax.experimental import pallas as pl
from jax.experimental.pallas import tpu as pltpu
from jax.experimental.pallas import tpu_sc as plsc

N = 8192
D = 256
K = 8192
BN = 2048
BK = 512
COMMITMENT_COST = 0.25
ENTROPY_WEIGHT = 0.1

_NC = 2
_NS = 16
_NW = _NC * _NS
_BPW = N // _NW


def _bits(x):
    return lax.bitcast_convert_type(x, jnp.int32)


def _flt(i):
    return lax.bitcast_convert_type(i, jnp.float32)


def _rev9(x):
    r = jnp.zeros_like(x)
    for b in range(9):
        r = r | (((x >> b) & 1) << (8 - b))
    return r


def _argmin_body(z_ref, w_ref, w2_ref, z2_ref, idx_ref):
    z = z_ref[...]
    z2 = z2_ref[...]
    pos = lax.broadcasted_iota(jnp.int32, (BK // 2, BN), 0)
    orig = _rev9(pos)
    origp1 = orig + 1

    def step(kb, carry):
        sv, mi = carry
        wb = w_ref[pl.ds(kb * BK, BK), :]
        m = lax.dot_general(wb, z, (((1,), (1,)), ((), ())),
                            preferred_element_type=jnp.float32)
        w2b = w2_ref[pl.ds(kb * BK, BK), :]
        d2 = (z2 + w2b) - 2.0 * m
        dist = jnp.sqrt(jnp.maximum(d2, 0.0))
        h = BK // 2
        c = dist[:h] <= dist[h:]
        tv = jnp.where(c, dist[:h], dist[h:])
        ti = jnp.where(c, orig, origp1)
        while tv.shape[0] > 8:
            h = tv.shape[0] // 2
            c = tv[:h] <= tv[h:]
            tv = jnp.where(c, tv[:h], tv[h:])
            ti = jnp.where(c, ti[:h], ti[h:])
        s = jnp.min(tv, axis=0, keepdims=True)
        bidx = jnp.min(jnp.where(tv == s, ti, K), axis=0,
                       keepdims=True) + kb * BK
        upd = s < sv
        return jnp.where(upd, s, sv), jnp.where(upd, bidx, mi)

    sv0 = jnp.full((1, BN), jnp.inf, jnp.float32)
    mi0 = jnp.zeros((1, BN), jnp.int32)
    _, mi = lax.fori_loop(0, K // BK, step, (sv0, mi0))
    idx_ref[...] = mi


_argmin_call = pl.pallas_call(
    _argmin_body,
    grid=(N // BN,),
    in_specs=[
        pl.BlockSpec((BN, D), lambda i: (i, 0)),
        pl.BlockSpec((K, D), lambda i: (0, 0)),
        pl.BlockSpec((K, 1), lambda i: (0, 0)),
        pl.BlockSpec((1, BN), lambda i: (0, i)),
    ],
    out_specs=pl.BlockSpec((1, BN), lambda i: (0, i)),
    out_shape=jax.ShapeDtypeStruct((1, N), jnp.int32),
)


_SCH = _BPW // 128


@functools.cache
def _make_gather_zq():
    mesh = plsc.VectorSubcoreMesh(core_axis_name="c", subcore_axis_name="s")

    @functools.partial(
        pl.kernel,
        mesh=mesh,
        out_type=(
            jax.ShapeDtypeStruct((N, D), jnp.float32),
            jax.ShapeDtypeStruct((_NC, K), jnp.float32),
        ),
        scratch_types=[
            pltpu.VMEM((_BPW,), jnp.int32),
            pltpu.VMEM((_BPW, D), jnp.float32),
            pltpu.VMEM((_SCH, 128), jnp.int32),
            pltpu.VMEM((128,), jnp.float32),
            pltpu.VMEM_SHARED((K,), jnp.float32),
            pltpu.SemaphoreType.DMA,
        ],
    )
    def _gather_zq(w_hbm, idx_hbm, zeros_hbm, out_hbm, cnt_hbm,
                   idx_v, rows_v, idxc_v, ones_v, counts_sh, sem):
        cid = lax.axis_index("c")
        sid = lax.axis_index("s")
        wid = sid * _NC + cid
        base = wid * _BPW
        pltpu.sync_copy(idx_hbm.at[pl.ds(base, _BPW)], idx_v)
        pltpu.async_copy(w_hbm.at[idx_v], rows_v, sem).wait()
        pltpu.sync_copy(rows_v, out_hbm.at[pl.ds(base, _BPW)])

        for j in range(_SCH):
            pltpu.sync_copy(idx_hbm.at[pl.ds(base + j * 128, 128)], idxc_v.at[j])
        for j in range(8):
            ones_v[pl.ds(j * 16, 16)] = jnp.full((16,), 1.0, jnp.float32)

        @pl.when(sid == 0)
        def _zero():
            pltpu.sync_copy(zeros_hbm.at[cid], counts_sh)

        plsc.subcore_barrier()
        for j in range(_SCH):
            pltpu.sync_copy(ones_v, counts_sh.at[idxc_v.at[j]], add=True)
        plsc.subcore_barrier()

        @pl.when(sid == 0)
        def _out():
            pltpu.sync_copy(counts_sh, cnt_hbm.at[cid])

    return _gather_zq


def _loss_body(ze_ref, zq_ref, cnt_ref, st_ref, scal_ref, acc_ref):
    i = pl.program_id(0)
    nsteps = pl.num_programs(0)

    @pl.when(i == 0)
    def _init():
        acc_ref[0] = 0.0

    ze = ze_ref[...]
    zq = zq_ref[...]
    diff = zq - ze
    st_ref[...] = ze + diff
    acc_ref[0] += jnp.sum(diff * diff)

    @pl.when(i == nsteps - 1)
    def _fin():
        cnt = cnt_ref[0, :] + cnt_ref[1, :]
        p = cnt * (1.0 / N) + 1e-10
        ent = -jnp.sum(p * jnp.log(p))
        cb = acc_ref[0] / (N * D)
        scal_ref[0] = cb
        scal_ref[1] = COMMITMENT_COST * cb
        scal_ref[2] = -ENTROPY_WEIGHT * (ent / jnp.log(float(K)))
        scal_ref[3] = jnp.exp(ent)


_loss_call = pl.pallas_call(
    _loss_body,
    grid=(N // BN,),
    in_specs=[
        pl.BlockSpec((BN, D), lambda i: (i, 0)),
        pl.BlockSpec((BN, D), lambda i: (i, 0)),
        pl.BlockSpec((_NC, K), lambda i: (0, 0)),
    ],
    out_specs=[
        pl.BlockSpec((BN, D), lambda i: (i, 0)),
        pl.BlockSpec(memory_space=pltpu.SMEM),
    ],
    out_shape=[
        jax.ShapeDtypeStruct((N, D), jnp.float32),
        jax.ShapeDtypeStruct((4,), jnp.float32),
    ],
    scratch_shapes=[
        pltpu.SMEM((1,), jnp.float32),
    ],
)


_BRTAB = np.array([int(format(p, "09b")[::-1], 2) for p in range(512)])
_BRPERM = (np.arange(K).reshape(-1, 512)[:, _BRTAB]).reshape(-1)


def kernel(z_e, W):
    z2 = jnp.sum(z_e * z_e, axis=-1)[None, :]
    W_br = W[_BRPERM]
    w2_br = jnp.sum(W_br * W_br, axis=-1, keepdims=True)
    idx_row = _argmin_call(z_e, W_br, w2_br, z2)
    idx = idx_row.reshape(N)
    zeros2k = jnp.zeros((_NC, K), jnp.float32)
    z_q, counts = _make_gather_zq()(W, idx, zeros2k)
    z_q_st, scal = _loss_call(z_e, z_q, counts)
    return (z_q_st, idx, scal[0], scal[1], scal[2], scal[3])


# speedup vs baseline: 1.3387x; 1.0634x over previous
import functools

import jax
import jax.numpy as jnp
import numpy as np
from jax import lax
from jax.experimental import pallas as pl
from jax.experimental.pallas import tpu as pltpu
from jax.experimental.pallas import tpu_sc as plsc

N = 8192
D = 256
K = 8192
BN = 2048
BK = 1024
_BKB = 10
COMMITMENT_COST = 0.25
ENTROPY_WEIGHT = 0.1

_NC = 2
_NS = 16
_NW = _NC * _NS
_BPW = N // _NW


def _bits(x):
    return lax.bitcast_convert_type(x, jnp.int32)


def _flt(i):
    return lax.bitcast_convert_type(i, jnp.float32)


def _rev(x):
    r = jnp.zeros_like(x)
    for b in range(_BKB):
        r = r | (((x >> b) & 1) << (_BKB - 1 - b))
    return r


def _argmin_body(z_ref, w_ref, w2_ref, z2_ref, idx_ref):
    z = z_ref[...]
    z2 = z2_ref[...]
    pos = lax.broadcasted_iota(jnp.int32, (BK // 2, BN), 0)
    orig = _rev(pos)
    origp1 = orig + 1

    def step(kb, carry):
        sv, mi = carry
        wb = w_ref[pl.ds(kb * BK, BK), :]
        m = lax.dot_general(wb, z, (((1,), (1,)), ((), ())),
                            preferred_element_type=jnp.float32)
        w2b = w2_ref[pl.ds(kb * BK, BK), :]
        d2 = (z2 + w2b) - 2.0 * m
        dist = jnp.sqrt(jnp.maximum(d2, 0.0))
        h = BK // 2
        c = dist[:h] <= dist[h:]
        tv = jnp.where(c, dist[:h], dist[h:])
        ti = jnp.where(c, orig, origp1)
        while tv.shape[0] > 8:
            h = tv.shape[0] // 2
            c = tv[:h] <= tv[h:]
            tv = jnp.where(c, tv[:h], tv[h:])
            ti = jnp.where(c, ti[:h], ti[h:])
        s = jnp.min(tv, axis=0, keepdims=True)
        bidx = jnp.min(jnp.where(tv == s, ti, K), axis=0,
                       keepdims=True) + kb * BK
        upd = s < sv
        return jnp.where(upd, s, sv), jnp.where(upd, bidx, mi)

    sv0 = jnp.full((1, BN), jnp.inf, jnp.float32)
    mi0 = jnp.zeros((1, BN), jnp.int32)
    _, mi = lax.fori_loop(0, K // BK, step, (sv0, mi0))
    idx_ref[...] = mi


_argmin_call = pl.pallas_call(
    _argmin_body,
    grid=(N // BN,),
    in_specs=[
        pl.BlockSpec((BN, D), lambda i: (i, 0)),
        pl.BlockSpec((K, D), lambda i: (0, 0)),
        pl.BlockSpec((K, 1), lambda i: (0, 0)),
        pl.BlockSpec((1, BN), lambda i: (0, i)),
    ],
    out_specs=pl.BlockSpec((1, BN), lambda i: (0, i)),
    out_shape=jax.ShapeDtypeStruct((1, N), jnp.int32),
)


_SCH = _BPW // 128


@functools.cache
def _make_gather_zq():
    mesh = plsc.VectorSubcoreMesh(core_axis_name="c", subcore_axis_name="s")

    @functools.partial(
        pl.kernel,
        mesh=mesh,
        out_type=(
            jax.ShapeDtypeStruct((N, D), jnp.float32),
            jax.ShapeDtypeStruct((_NC, K), jnp.float32),
        ),
        scratch_types=[
            pltpu.VMEM((_BPW,), jnp.int32),
            pltpu.VMEM((_BPW, D), jnp.float32),
            pltpu.VMEM((_SCH, 128), jnp.int32),
            pltpu.VMEM((128,), jnp.float32),
            pltpu.VMEM_SHARED((K,), jnp.float32),
            pltpu.SemaphoreType.DMA,
        ],
    )
    def _gather_zq(w_hbm, idx_hbm, zeros_hbm, out_hbm, cnt_hbm,
                   idx_v, rows_v, idxc_v, ones_v, counts_sh, sem):
        cid = lax.axis_index("c")
        sid = lax.axis_index("s")
        wid = sid * _NC + cid
        base = wid * _BPW
        pltpu.sync_copy(idx_hbm.at[pl.ds(base, _BPW)], idx_v)
        pltpu.async_copy(w_hbm.at[idx_v], rows_v, sem).wait()
        pltpu.sync_copy(rows_v, out_hbm.at[pl.ds(base, _BPW)])

        for j in range(_SCH):
            pltpu.sync_copy(idx_hbm.at[pl.ds(base + j * 128, 128)], idxc_v.at[j])
        for j in range(8):
            ones_v[pl.ds(j * 16, 16)] = jnp.full((16,), 1.0, jnp.float32)

        @pl.when(sid == 0)
        def _zero():
            pltpu.sync_copy(zeros_hbm.at[cid], counts_sh)

        plsc.subcore_barrier()
        for j in range(_SCH):
            pltpu.sync_copy(ones_v, counts_sh.at[idxc_v.at[j]], add=True)
        plsc.subcore_barrier()

        @pl.when(sid == 0)
        def _out():
            pltpu.sync_copy(counts_sh, cnt_hbm.at[cid])

    return _gather_zq


def _loss_body(ze_ref, zq_ref, cnt_ref, st_ref, scal_ref, acc_ref):
    i = pl.program_id(0)
    nsteps = pl.num_programs(0)

    @pl.when(i == 0)
    def _init():
        acc_ref[0] = 0.0

    ze = ze_ref[...]
    zq = zq_ref[...]
    diff = zq - ze
    st_ref[...] = ze + diff
    acc_ref[0] += jnp.sum(diff * diff)

    @pl.when(i == nsteps - 1)
    def _fin():
        cnt = cnt_ref[0, :] + cnt_ref[1, :]
        p = cnt * (1.0 / N) + 1e-10
        ent = -jnp.sum(p * jnp.log(p))
        cb = acc_ref[0] / (N * D)
        scal_ref[0] = cb
        scal_ref[1] = COMMITMENT_COST * cb
        scal_ref[2] = -ENTROPY_WEIGHT * (ent / jnp.log(float(K)))
        scal_ref[3] = jnp.exp(ent)


_loss_call = pl.pallas_call(
    _loss_body,
    grid=(N // BN,),
    in_specs=[
        pl.BlockSpec((BN, D), lambda i: (i, 0)),
        pl.BlockSpec((BN, D), lambda i: (i, 0)),
        pl.BlockSpec((_NC, K), lambda i: (0, 0)),
    ],
    out_specs=[
        pl.BlockSpec((BN, D), lambda i: (i, 0)),
        pl.BlockSpec(memory_space=pltpu.SMEM),
    ],
    out_shape=[
        jax.ShapeDtypeStruct((N, D), jnp.float32),
        jax.ShapeDtypeStruct((4,), jnp.float32),
    ],
    scratch_shapes=[
        pltpu.SMEM((1,), jnp.float32),
    ],
)


_BRTAB = np.array([int(format(p, f"0{_BKB}b")[::-1], 2) for p in range(BK)])
_BRPERM = (np.arange(K).reshape(-1, BK)[:, _BRTAB]).reshape(-1)


def kernel(z_e, W):
    z2 = jnp.sum(z_e * z_e, axis=-1)[None, :]
    W_br = W[_BRPERM]
    w2_br = jnp.sum(W_br * W_br, axis=-1, keepdims=True)
    idx_row = _argmin_call(z_e, W_br, w2_br, z2)
    idx = idx_row.reshape(N)
    zeros2k = jnp.zeros((_NC, K), jnp.float32)
    z_q, counts = _make_gather_zq()(W, idx, zeros2k)
    z_q_st, scal = _loss_call(z_e, z_q, counts)
    return (z_q_st, idx, scal[0], scal[1], scal[2], scal[3])


# speedup vs baseline: 1.3847x; 1.0344x over previous
import functools

import jax
import jax.numpy as jnp
import numpy as np
from jax import lax
from jax.experimental import pallas as pl
from jax.experimental.pallas import tpu as pltpu
from jax.experimental.pallas import tpu_sc as plsc

N = 8192
D = 256
K = 8192
BN = 2048
BK = 2048
_BKB = 11
COMMITMENT_COST = 0.25
ENTROPY_WEIGHT = 0.1

_NC = 2
_NS = 16
_NW = _NC * _NS
_BPW = N // _NW


def _bits(x):
    return lax.bitcast_convert_type(x, jnp.int32)


def _flt(i):
    return lax.bitcast_convert_type(i, jnp.float32)


def _rev(x):
    r = jnp.zeros_like(x)
    for b in range(_BKB):
        r = r | (((x >> b) & 1) << (_BKB - 1 - b))
    return r


def _argmin_body(z_ref, w_ref, w2_ref, z2_ref, idx_ref):
    z = z_ref[...]
    z2 = z2_ref[...]
    pos = lax.broadcasted_iota(jnp.int32, (BK // 2, BN), 0)
    orig = _rev(pos)
    origp1 = orig + 1

    def step(kb, carry):
        sv, mi = carry
        wb = w_ref[pl.ds(kb * BK, BK), :]
        m = lax.dot_general(wb, z, (((1,), (1,)), ((), ())),
                            preferred_element_type=jnp.float32)
        w2b = w2_ref[pl.ds(kb * BK, BK), :]
        d2 = (z2 + w2b) - 2.0 * m
        dist = jnp.sqrt(jnp.maximum(d2, 0.0))
        h = BK // 2
        c = dist[:h] <= dist[h:]
        tv = jnp.where(c, dist[:h], dist[h:])
        ti = jnp.where(c, orig, origp1)
        while tv.shape[0] > 8:
            h = tv.shape[0] // 2
            c = tv[:h] <= tv[h:]
            tv = jnp.where(c, tv[:h], tv[h:])
            ti = jnp.where(c, ti[:h], ti[h:])
        s = jnp.min(tv, axis=0, keepdims=True)
        bidx = jnp.min(jnp.where(tv == s, ti, K), axis=0,
                       keepdims=True) + kb * BK
        upd = s < sv
        return jnp.where(upd, s, sv), jnp.where(upd, bidx, mi)

    sv0 = jnp.full((1, BN), jnp.inf, jnp.float32)
    mi0 = jnp.zeros((1, BN), jnp.int32)
    _, mi = lax.fori_loop(0, K // BK, step, (sv0, mi0))
    idx_ref[...] = mi


_argmin_call = pl.pallas_call(
    _argmin_body,
    grid=(N // BN,),
    in_specs=[
        pl.BlockSpec((BN, D), lambda i: (i, 0)),
        pl.BlockSpec((K, D), lambda i: (0, 0)),
        pl.BlockSpec((K, 1), lambda i: (0, 0)),
        pl.BlockSpec((1, BN), lambda i: (0, i)),
    ],
    out_specs=pl.BlockSpec((1, BN), lambda i: (0, i)),
    out_shape=jax.ShapeDtypeStruct((1, N), jnp.int32),
)


_SCH = _BPW // 128


@functools.cache
def _make_gather_zq():
    mesh = plsc.VectorSubcoreMesh(core_axis_name="c", subcore_axis_name="s")

    @functools.partial(
        pl.kernel,
        mesh=mesh,
        out_type=(
            jax.ShapeDtypeStruct((N, D), jnp.float32),
            jax.ShapeDtypeStruct((_NC, K), jnp.float32),
        ),
        scratch_types=[
            pltpu.VMEM((_BPW,), jnp.int32),
            pltpu.VMEM((_BPW, D), jnp.float32),
            pltpu.VMEM((_SCH, 128), jnp.int32),
            pltpu.VMEM((128,), jnp.float32),
            pltpu.VMEM_SHARED((K,), jnp.float32),
            pltpu.SemaphoreType.DMA,
        ],
    )
    def _gather_zq(w_hbm, idx_hbm, zeros_hbm, out_hbm, cnt_hbm,
                   idx_v, rows_v, idxc_v, ones_v, counts_sh, sem):
        cid = lax.axis_index("c")
        sid = lax.axis_index("s")
        wid = sid * _NC + cid
        base = wid * _BPW
        pltpu.sync_copy(idx_hbm.at[pl.ds(base, _BPW)], idx_v)
        pltpu.async_copy(w_hbm.at[idx_v], rows_v, sem).wait()
        pltpu.sync_copy(rows_v, out_hbm.at[pl.ds(base, _BPW)])

        for j in range(_SCH):
            pltpu.sync_copy(idx_hbm.at[pl.ds(base + j * 128, 128)], idxc_v.at[j])
        for j in range(8):
            ones_v[pl.ds(j * 16, 16)] = jnp.full((16,), 1.0, jnp.float32)

        @pl.when(sid == 0)
        def _zero():
            pltpu.sync_copy(zeros_hbm.at[cid], counts_sh)

        plsc.subcore_barrier()
        for j in range(_SCH):
            pltpu.sync_copy(ones_v, counts_sh.at[idxc_v.at[j]], add=True)
        plsc.subcore_barrier()

        @pl.when(sid == 0)
        def _out():
            pltpu.sync_copy(counts_sh, cnt_hbm.at[cid])

    return _gather_zq


def _loss_body(ze_ref, zq_ref, cnt_ref, st_ref, scal_ref, acc_ref):
    i = pl.program_id(0)
    nsteps = pl.num_programs(0)

    @pl.when(i == 0)
    def _init():
        acc_ref[0] = 0.0

    ze = ze_ref[...]
    zq = zq_ref[...]
    diff = zq - ze
    st_ref[...] = ze + diff
    acc_ref[0] += jnp.sum(diff * diff)

    @pl.when(i == nsteps - 1)
    def _fin():
        cnt = cnt_ref[0, :] + cnt_ref[1, :]
        p = cnt * (1.0 / N) + 1e-10
        ent = -jnp.sum(p * jnp.log(p))
        cb = acc_ref[0] / (N * D)
        scal_ref[0] = cb
        scal_ref[1] = COMMITMENT_COST * cb
        scal_ref[2] = -ENTROPY_WEIGHT * (ent / jnp.log(float(K)))
        scal_ref[3] = jnp.exp(ent)


_loss_call = pl.pallas_call(
    _loss_body,
    grid=(N // BN,),
    in_specs=[
        pl.BlockSpec((BN, D), lambda i: (i, 0)),
        pl.BlockSpec((BN, D), lambda i: (i, 0)),
        pl.BlockSpec((_NC, K), lambda i: (0, 0)),
    ],
    out_specs=[
        pl.BlockSpec((BN, D), lambda i: (i, 0)),
        pl.BlockSpec(memory_space=pltpu.SMEM),
    ],
    out_shape=[
        jax.ShapeDtypeStruct((N, D), jnp.float32),
        jax.ShapeDtypeStruct((4,), jnp.float32),
    ],
    scratch_shapes=[
        pltpu.SMEM((1,), jnp.float32),
    ],
)


_BRTAB = np.array([int(format(p, f"0{_BKB}b")[::-1], 2) for p in range(BK)])
_BRPERM = (np.arange(K).reshape(-1, BK)[:, _BRTAB]).reshape(-1)


def kernel(z_e, W):
    z2 = jnp.sum(z_e * z_e, axis=-1)[None, :]
    W_br = W[_BRPERM]
    w2_br = jnp.sum(W_br * W_br, axis=-1, keepdims=True)
    idx_row = _argmin_call(z_e, W_br, w2_br, z2)
    idx = idx_row.reshape(N)
    zeros2k = jnp.zeros((_NC, K), jnp.float32)
    z_q, counts = _make_gather_zq()(W, idx, zeros2k)
    z_q_st, scal = _loss_call(z_e, z_q, counts)
    return (z_q_st, idx, scal[0], scal[1], scal[2], scal[3])
